# baseline (speedup 1.0000x reference)
import functools

import jax
import jax.numpy as jnp
from jax import lax
from jax.experimental import pallas as pl
from jax.experimental.pallas import tpu as pltpu
from jax.experimental.pallas import tpu_sc as plsc

_N = 16384
_EMB = 64
_NC = 2
_NS = 16
_NW = _NC * _NS
_BPW = _N // _NW
_CHUNK = 128
_NCHUNK = _BPW // _CHUNK


def _sc_gather(s, r, o, entity_table, relation_table):
  mesh = plsc.VectorSubcoreMesh(
      core_axis_name="c", subcore_axis_name="s",
      num_cores=_NC, num_subcores=_NS)

  @functools.partial(
      pl.kernel,
      out_type=[jax.ShapeDtypeStruct((_N, _EMB), jnp.float32)] * 3,
      mesh=mesh,
      compiler_params=pltpu.CompilerParams(use_tc_tiling_on_sc=False),
      scratch_types=[
          pltpu.VMEM((_BPW,), jnp.int32),
          pltpu.VMEM((_BPW,), jnp.int32),
          pltpu.VMEM((_BPW,), jnp.int32),
          pltpu.VMEM((_BPW, _EMB), jnp.float32),
          pltpu.VMEM((_BPW, _EMB), jnp.float32),
          pltpu.VMEM((_BPW, _EMB), jnp.float32),
          pltpu.SemaphoreType.DMA,
      ],
  )
  def k(s_h, r_h, o_h, ent_h, rel_h, xs_h, xr_h, xo_h,
        si_v, ri_v, oi_v, xs_v, xr_v, xo_v, sem):
    wid = lax.axis_index("s") * _NC + lax.axis_index("c")
    base = wid * _BPW
    pltpu.sync_copy(s_h.at[pl.ds(base, _BPW)], si_v)
    pltpu.sync_copy(r_h.at[pl.ds(base, _BPW)], ri_v)
    pltpu.sync_copy(o_h.at[pl.ds(base, _BPW)], oi_v)
    copies = []
    for j in range(_NCHUNK):
      sl = pl.ds(j * _CHUNK, _CHUNK)
      copies.append(pltpu.async_copy(ent_h.at[si_v.at[sl]], xs_v.at[sl], sem))
      copies.append(pltpu.async_copy(rel_h.at[ri_v.at[sl]], xr_v.at[sl], sem))
      copies.append(pltpu.async_copy(ent_h.at[oi_v.at[sl]], xo_v.at[sl], sem))
    for c in copies:
      c.wait()
    out_sl = pl.ds(base, _BPW)
    pltpu.sync_copy(xs_v, xs_h.at[out_sl])
    pltpu.sync_copy(xr_v, xr_h.at[out_sl])
    pltpu.sync_copy(xo_v, xo_h.at[out_sl])

  return k(s, r, o, entity_table, relation_table)


_BLK = 2048


def _tc_body(xs_ref, xr_ref, xo_ref, w_ref, b_ref, out_ref):
  w = w_ref[...]
  b = b_ref[...]
  out_ref[:, 0:_EMB] = (
      jnp.dot(xs_ref[...], w, preferred_element_type=jnp.float32) + b)
  out_ref[:, _EMB:2 * _EMB] = (
      jnp.dot(xr_ref[...], w, preferred_element_type=jnp.float32) + b)
  out_ref[:, 2 * _EMB:3 * _EMB] = (
      jnp.dot(xo_ref[...], w, preferred_element_type=jnp.float32) + b)


def _tc_project(xs, xr, xo, W, b):
  row_spec = pl.BlockSpec((_BLK, _EMB), lambda i: (i, 0))
  return pl.pallas_call(
      _tc_body,
      grid=(_N // _BLK,),
      in_specs=[
          row_spec, row_spec, row_spec,
          pl.BlockSpec((_EMB, _EMB), lambda i: (0, 0)),
          pl.BlockSpec((1, _EMB), lambda i: (0, 0)),
      ],
      out_specs=pl.BlockSpec((_BLK, 3 * _EMB), lambda i: (i, 0)),
      out_shape=jax.ShapeDtypeStruct((_N, 3 * _EMB), jnp.float32),
  )(xs, xr, xo, W, b.reshape(1, _EMB))


def kernel(s, r, o, entity_table, relation_table, W, b):
  s = s.astype(jnp.int32)
  r = r.astype(jnp.int32)
  o = o.astype(jnp.int32)
  xs, xr, xo = _sc_gather(s, r, o, entity_table, relation_table)
  return _tc_project(xs, xr, xo, W, b)
